# baseline (speedup 1.0000x reference)
import functools
import math

import jax
import jax.numpy as jnp
from jax import lax
from jax.experimental import pallas as pl
from jax.experimental.pallas import tpu as pltpu
from jax.experimental.pallas import tpu_sc as plsc

NC = 2
NS = 16
NW = NC * NS
LANES = 16
CHUNK = 128
PADW = 128
TBLK = 256
NBUF = 4


@functools.partial(jax.jit, static_argnums=(2,))
def _prep(table_t, remp, scale):
    d, v = table_t.shape
    nfull = v // TBLK
    vrem = v - nfull * TBLK
    niter = (nfull + NW - 1) // NW

    @functools.partial(
        pl.kernel,
        mesh=plsc.VectorSubcoreMesh(core_axis_name="c", subcore_axis_name="s"),
        compiler_params=pltpu.CompilerParams(needs_layout_passes=False),
        out_type=jax.ShapeDtypeStruct((v, PADW), jnp.float32),
        scratch_types=[
            pltpu.VMEM((2, d, TBLK + 1), jnp.float32),
            pltpu.VMEM((2, TBLK, PADW), jnp.float32),
            pltpu.SemaphoreType.DMA,
            pltpu.SemaphoreType.DMA,
            pltpu.SemaphoreType.DMA,
            pltpu.SemaphoreType.DMA,
        ],
    )
    def body(tt_hbm, remp_hbm, scr_hbm, bin_v, bout_v, si0, si1, so0, so1):
        wid = lax.axis_index("s") * NC + lax.axis_index("c")
        sin = (si0, si1)
        sout = (so0, so1)
        iot = lax.iota(jnp.int32, LANES)

        def blk_of(t):
            return t * NW + wid

        def get_in(t, b):
            @pl.when(blk_of(t) < nfull)
            def _():
                pltpu.async_copy(
                    tt_hbm.at[:, pl.ds(blk_of(t) * TBLK, TBLK)],
                    bin_v.at[b, :, pl.ds(0, TBLK)],
                    sin[b],
                )

        def wait_in(t, b):
            @pl.when(blk_of(t) < nfull)
            def _():
                pltpu.make_async_copy(
                    tt_hbm.at[:, pl.ds(blk_of(t) * TBLK, TBLK)],
                    bin_v.at[b, :, pl.ds(0, TBLK)],
                    sin[b],
                ).wait()

        def put_out(t, b):
            @pl.when(blk_of(t) < nfull)
            def _():
                pltpu.async_copy(
                    bout_v.at[b],
                    scr_hbm.at[pl.ds(blk_of(t) * TBLK, TBLK)],
                    sout[b],
                )

        def wait_out(t, b):
            @pl.when(blk_of(t) < nfull)
            def _():
                pltpu.make_async_copy(
                    bout_v.at[b],
                    scr_hbm.at[pl.ds(blk_of(t) * TBLK, TBLK)],
                    sout[b],
                ).wait()

        def transpose_block(t, b):
            @pl.when(blk_of(t) < nfull)
            def _():
                @plsc.parallel_loop(0, TBLK, unroll=8)
                def trow(r):
                    cols = jnp.full((LANES,), r, jnp.int32)
                    for g in range(d // LANES):
                        rows = iot + g * LANES
                        val = plsc.load_gather(bin_v.at[b], [rows, cols])
                        bout_v[b, r, pl.ds(g * LANES, LANES)] = val * scale

        @pl.when(wid == NW - 1)
        def _():
            pltpu.sync_copy(remp_hbm, bout_v.at[0, : remp_hbm.shape[0]])

            def rrow(r, carry):
                for g in range(d // LANES):
                    s = pl.ds(g * LANES, LANES)
                    bout_v[0, r, s] = bout_v[0, r, s] * scale
                return carry

            lax.fori_loop(0, remp_hbm.shape[0], rrow, 0)
            pltpu.sync_copy(
                bout_v.at[0, : remp_hbm.shape[0]],
                scr_hbm.at[pl.ds(nfull * TBLK, vrem)],
            )

        get_in(0, 0)

        def pair_body(p, carry):
            t0 = p * 2
            for b in range(2):
                t = t0 + b

                @pl.when(t + 1 < niter)
                def _(t=t, b=b):
                    get_in(t + 1, 1 - b)

                wait_in(t, b)

                @pl.when(t >= 2)
                def _(t=t, b=b):
                    wait_out(t - 2, b)

                transpose_block(t, b)
                put_out(t, b)
            return carry

        t_end = 2 * ((niter + 1) // 2)
        lax.fori_loop(0, t_end // 2, pair_body, 0)
        wait_out(t_end - 2, (t_end - 2) % 2)
        wait_out(t_end - 1, (t_end - 1) % 2)

    return body(table_t, remp)


@functools.partial(jax.jit, static_argnums=(2,))
def _embed(idx, scr, n_chunks):
    total = NW * n_chunks * CHUNK

    @functools.partial(
        pl.kernel,
        mesh=plsc.VectorSubcoreMesh(core_axis_name="c", subcore_axis_name="s"),
        out_type=jax.ShapeDtypeStruct((total, PADW), jnp.float32),
        scratch_types=[
            pltpu.VMEM((n_chunks, CHUNK), jnp.int32),
            pltpu.VMEM((NBUF, CHUNK, PADW), jnp.float32),
            [pltpu.SemaphoreType.DMA] * NBUF,
            [pltpu.SemaphoreType.DMA] * NBUF,
        ],
    )
    def body(idx_hbm, scr_hbm, out_hbm, idx_v, rows_v, sin, sout):
        wid = lax.axis_index("s") * NC + lax.axis_index("c")
        pltpu.sync_copy(idx_hbm.at[pl.ds(wid * n_chunks, n_chunks)], idx_v)
        base = wid * n_chunks * CHUNK

        def gather(j, b):
            pltpu.async_copy(scr_hbm.at[idx_v.at[j]], rows_v.at[b], sin[b])

        def wait_gather(j, b):
            pltpu.make_async_copy(
                scr_hbm.at[idx_v.at[j]], rows_v.at[b], sin[b]
            ).wait()

        def putout(j, b):
            pltpu.async_copy(
                rows_v.at[b], out_hbm.at[pl.ds(base + j * CHUNK, CHUNK)], sout[b]
            )

        def wait_putout(j, b):
            pltpu.make_async_copy(
                rows_v.at[b], out_hbm.at[pl.ds(base + j * CHUNK, CHUNK)], sout[b]
            ).wait()

        for b in range(NBUF - 1):
            gather(b, b)

        def ring_body(q, carry):
            j0 = q * NBUF
            for b in range(NBUF):
                j = j0 + b
                wait_gather(j, b)
                putout(j, b)

                @pl.when(j + NBUF - 1 < n_chunks)
                def _(j=j, b=b):
                    nb = (b - 1) % NBUF

                    @pl.when(j >= 1)
                    def _():
                        wait_putout(j - 1, nb)

                    gather(j + NBUF - 1, nb)

            return carry

        lax.fori_loop(0, n_chunks // NBUF, ring_body, 0)
        for k in range(NBUF):
            j = n_chunks - NBUF + k
            wait_putout(j, j % NBUF)

    return body(idx, scr)


def kernel(x, table):
    b, s = x.shape
    v, d = table.shape
    total = b * s
    assert total % (NW * CHUNK) == 0
    n_chunks = total // (NW * CHUNK)
    idx = x.reshape(NW * n_chunks, CHUNK).astype(jnp.int32)
    scale = float(math.sqrt(d))
    vrem = v % TBLK
    remp = jnp.pad(table[v - vrem :], ((0, 0), (0, PADW - d)))
    scr = _prep(table.T, remp, scale)
    out = _embed(idx, scr, n_chunks)
    return out[:, :d].reshape(b, s, d)


# speedup vs baseline: 1.2648x; 1.2648x over previous
import functools
import math

import jax
import jax.numpy as jnp
from jax import lax
from jax.experimental import pallas as pl
from jax.experimental.pallas import tpu as pltpu
from jax.experimental.pallas import tpu_sc as plsc

NC = 2
NS = 16
NW = NC * NS
LANES = 16
CHUNK = 128
PADW = 128
NBUF = 4


@functools.partial(jax.jit, static_argnums=(2, 3))
def _embed(idx, table, n_chunks, scale):
    total = NW * n_chunks * CHUNK
    d = table.shape[1]

    @functools.partial(
        pl.kernel,
        mesh=plsc.VectorSubcoreMesh(core_axis_name="c", subcore_axis_name="s"),
        out_type=jax.ShapeDtypeStruct((total, d), jnp.float32),
        scratch_types=[
            pltpu.VMEM((n_chunks, CHUNK), jnp.int32),
            pltpu.VMEM((NBUF, CHUNK, d), jnp.float32),
            [pltpu.SemaphoreType.DMA] * NBUF,
            [pltpu.SemaphoreType.DMA] * NBUF,
        ],
    )
    def body(idx_hbm, table_hbm, out_hbm, idx_v, rows_v, sin, sout):
        wid = lax.axis_index("s") * NC + lax.axis_index("c")
        pltpu.sync_copy(idx_hbm.at[pl.ds(wid * n_chunks, n_chunks)], idx_v)
        base = wid * n_chunks * CHUNK

        def gather(j, b):
            pltpu.async_copy(table_hbm.at[idx_v.at[j]], rows_v.at[b], sin[b])

        def wait_gather(j, b):
            pltpu.make_async_copy(
                table_hbm.at[idx_v.at[j]], rows_v.at[b], sin[b]
            ).wait()

        def putout(j, b):
            pltpu.async_copy(
                rows_v.at[b], out_hbm.at[pl.ds(base + j * CHUNK, CHUNK)], sout[b]
            )

        def wait_putout(j, b):
            pltpu.make_async_copy(
                rows_v.at[b], out_hbm.at[pl.ds(base + j * CHUNK, CHUNK)], sout[b]
            ).wait()

        def scale_buf(b):
            @plsc.parallel_loop(0, CHUNK, unroll=8)
            def scale_row(r):
                for c in range(64 // LANES):
                    s = pl.ds(c * LANES, LANES)
                    rows_v[b, r, s] = rows_v[b, r, s] * scale

        for b in range(NBUF - 1):
            gather(b, b)

        def ring_body(q, carry):
            j0 = q * NBUF
            for b in range(NBUF):
                j = j0 + b
                wait_gather(j, b)
                scale_buf(b)
                putout(j, b)

                @pl.when(j + NBUF - 1 < n_chunks)
                def _(j=j, b=b):
                    nb = (b - 1) % NBUF

                    @pl.when(j >= 1)
                    def _():
                        wait_putout(j - 1, nb)

                    gather(j + NBUF - 1, nb)

            return carry

        lax.fori_loop(0, n_chunks // NBUF, ring_body, 0)
        for k in range(NBUF):
            j = n_chunks - NBUF + k
            wait_putout(j, j % NBUF)

    return body(idx, table)


def kernel(x, table):
    b, s = x.shape
    v, d = table.shape
    total = b * s
    assert total % (NW * CHUNK) == 0
    n_chunks = total // (NW * CHUNK)
    idx = x.reshape(NW * n_chunks, CHUNK).astype(jnp.int32)
    tablep = jnp.pad(table, ((0, 0), (0, PADW - d)))
    out = _embed(idx, tablep, n_chunks, float(math.sqrt(d)))
    return out[:, :d].reshape(b, s, d)
